# baseline (speedup 1.0000x reference)
import functools
import math

import jax
import jax.numpy as jnp
from jax import lax
from jax.experimental import pallas as pl
from jax.experimental.pallas import tpu as pltpu
from jax.experimental.pallas import tpu_sc as plsc

_B = 64
_NT = 800
_NA = 3
_THR = 4.0
_BAL = (4.0, 1.0, 0.4)
_DIMS = ((80, 80), (40, 40), (20, 20))
_STRIDE = (8.0, 16.0, 32.0)
_ANCHORS_RAW = ((10.0, 13.0, 16.0, 30.0, 33.0, 23.0),
                (30.0, 61.0, 62.0, 45.0, 59.0, 119.0),
                (116.0, 90.0, 156.0, 198.0, 373.0, 326.0))
_ANCH = tuple(
    tuple((_ANCHORS_RAW[i][2 * a] / _STRIDE[i], _ANCHORS_RAW[i][2 * a + 1] / _STRIDE[i])
          for a in range(_NA))
    for i in range(3))

_NW = 32
_LANES = 16
_TV = _NT // _LANES
_TASKS = 5 * _NA * _TV
_KMAX = (_TASKS + _NW - 1) // _NW

_G = 10


def _dense_body(p0_ref, p1_ref, p2_ref, s_ref):
    @pl.when(pl.program_id(0) == 0)
    def _init():
        s_ref[0] = 0.0
        s_ref[1] = 0.0
        s_ref[2] = 0.0

    for idx, ref in enumerate((p0_ref, p1_ref, p2_ref)):
        x = ref[...]
        rows = x.shape[0]
        rid = lax.broadcasted_iota(jnp.int32, (rows, 128), 0)
        lid = lax.broadcasted_iota(jnp.int32, (rows, 128), 1)
        m = ((2 * (rid % 3) + lid) % 6) == 4
        sp = jnp.maximum(x, 0.0) + jnp.log(1.0 + jnp.exp(-jnp.abs(x)))
        s_ref[idx] += jnp.sum(jnp.where(m, sp, 0.0))


def _dense_sums(p0, p1, p2):
    r0 = p0.reshape(-1, 128)
    r1 = p1.reshape(-1, 128)
    r2 = p2.reshape(-1, 128)
    rb0, rb1, rb2 = r0.shape[0] // _G, r1.shape[0] // _G, r2.shape[0] // _G
    return pl.pallas_call(
        _dense_body,
        grid=(_G,),
        in_specs=[
            pl.BlockSpec((rb0, 128), lambda g: (g, 0)),
            pl.BlockSpec((rb1, 128), lambda g: (g, 0)),
            pl.BlockSpec((rb2, 128), lambda g: (g, 0)),
        ],
        out_specs=pl.BlockSpec(memory_space=pltpu.SMEM),
        out_shape=jax.ShapeDtypeStruct((3,), jnp.float32),
    )(r0, r1, r2)


def _atan_pos(z):
    inv = z > 1.0
    zz = jnp.where(inv, 1.0 / z, z)
    u = zz * zz
    p = 0.99997726 + u * (-0.33262347 + u * (0.19354346 + u * (
        -0.11643287 + u * (0.05265332 + u * (-0.01172120)))))
    a = zz * p
    return jnp.where(inv, (math.pi / 2) - a, a)


def _sigmoid(z):
    return 1.0 / (1.0 + jnp.exp(-z))


def _sc_body(tt_hbm, p0_hbm, p1_hbm, p2_hbm, out_hbm,
             tt_v, row_v, val_v, tbx_v, tby_v, tbw_v, tbh_v,
             psa_v, psb_v, psc_v, psd_v, pse_v, stage_v, sem):
    ps_v = (psa_v, psb_v, psc_v, psd_v, pse_v)
    wid = lax.axis_index("s") * 2 + lax.axis_index("c")
    pltpu.sync_copy(tt_hbm, tt_v)
    lanes = lax.iota(jnp.int32, 16)
    flanes = lanes.astype(jnp.float32)

    def task_decode(k):
        task = wid + _NW * k
        offi = jnp.minimum(lax.div(task, _NA * _TV), 4)
        rem = lax.rem(task, _NA * _TV)
        ai = lax.div(rem, _TV)
        tv = lax.rem(rem, _TV)
        padf = jnp.where(task >= _TASKS, 1.0, 0.0)
        return task, offi, ai, tv, padf

    def sel_anchor(lvl, ai):
        aw = jnp.where(ai == 0, _ANCH[lvl][0][0],
                       jnp.where(ai == 1, _ANCH[lvl][1][0], _ANCH[lvl][2][0]))
        ah = jnp.where(ai == 0, _ANCH[lvl][0][1],
                       jnp.where(ai == 1, _ANCH[lvl][1][1], _ANCH[lvl][2][1]))
        return aw, ah

    for lvl, (p_hbm, (hh, ww)) in enumerate(
            zip((p0_hbm, p1_hbm, p2_hbm), _DIMS)):
        fw = float(ww)
        fh = float(hh)

        def phase1(k, carry, _fw=fw, _fh=fh, _lvl=lvl):
            task, offi, ai, tv, padf = task_decode(k)
            base = tv * _LANES
            bi = tt_v[0, pl.ds(base, 16)]
            x = tt_v[2, pl.ds(base, 16)]
            y = tt_v[3, pl.ds(base, 16)]
            w = tt_v[4, pl.ds(base, 16)]
            h = tt_v[5, pl.ds(base, 16)]
            tx = x * _fw
            ty = y * _fh
            tw = w * _fw
            th = h * _fh
            aw, ah = sel_anchor(_lvl, ai)
            rw = tw / aw
            rh = th / ah
            rj = jnp.maximum(jnp.maximum(rw, 1.0 / rw),
                             jnp.maximum(rh, 1.0 / rh)) < _THR
            jjf = jnp.where((lax.rem(tx, 1.0) < 0.5) & (tx > 1.0), 1.0, 0.0)
            kkf = jnp.where((lax.rem(ty, 1.0) < 0.5) & (ty > 1.0), 1.0, 0.0)
            gxi = _fw - tx
            gyi = _fh - ty
            llf = jnp.where((lax.rem(gxi, 1.0) < 0.5) & (gxi > 1.0), 1.0, 0.0)
            mmf = jnp.where((lax.rem(gyi, 1.0) < 0.5) & (gyi > 1.0), 1.0, 0.0)
            e0 = jnp.where(offi == 0, 1.0, 0.0)
            e1 = jnp.where(offi == 1, 1.0, 0.0)
            e2 = jnp.where(offi == 2, 1.0, 0.0)
            e3 = jnp.where(offi == 3, 1.0, 0.0)
            e4 = jnp.where(offi == 4, 1.0, 0.0)
            jmf = e0 + e1 * jjf + e2 * kkf + e3 * llf + e4 * mmf
            rjf = jnp.where(rj, 1.0, 0.0)
            validf = jmf * rjf * (1.0 - padf)
            offx = jnp.where(offi == 1, 0.5, 0.0) + jnp.where(offi == 3, -0.5, 0.0)
            offy = jnp.where(offi == 2, 0.5, 0.0) + jnp.where(offi == 4, -0.5, 0.0)
            gxf = tx - offx
            gyf = ty - offy
            gijx = gxf.astype(jnp.int32)
            gijy = gyf.astype(jnp.int32)
            gi = jnp.clip(gijx, 0, ww - 1)
            gj = jnp.clip(gijy, 0, hh - 1)
            b = bi.astype(jnp.int32)
            row = ((b * _NA + ai) * hh + gj) * ww + gi
            row_v[k] = row
            val_v[k] = validf
            tbx_v[k] = tx - gijx.astype(jnp.float32)
            tby_v[k] = ty - gijy.astype(jnp.float32)
            tbw_v[k] = tw
            tbh_v[k] = th
            return carry

        lax.fori_loop(0, _KMAX, phase1, 0)

        copies = []
        for k in range(_KMAX):
            idx6 = row_v[k] * 6
            for c in range(5):
                copies.append(pltpu.make_async_copy(
                    p_hbm.at[idx6 + c], ps_v[c].at[k], sem))
        for cp in copies:
            cp.start()
        for cp in copies:
            cp.wait()

        def phase3(k, carry, _lvl=lvl):
            l_acc, c_acc, o_acc = carry
            task, offi, ai, tv, padf = task_decode(k)
            ps = [ps_v[c][k] for c in range(5)]
            validf = val_v[k]
            tbx = tbx_v[k]
            tby = tby_v[k]
            tbw = tbw_v[k]
            tbh = tbh_v[k]
            aw, ah = sel_anchor(_lvl, ai)
            pxx = _sigmoid(ps[0]) * 2.0 - 0.5
            pxy = _sigmoid(ps[1]) * 2.0 - 0.5
            sw = _sigmoid(ps[2]) * 2.0
            sh = _sigmoid(ps[3]) * 2.0
            pww = sw * sw * aw
            phh = sh * sh * ah
            eps = 1e-9
            b1x1 = pxx - pww * 0.5
            b1x2 = pxx + pww * 0.5
            b1y1 = pxy - phh * 0.5
            b1y2 = pxy + phh * 0.5
            b2x1 = tbx - tbw * 0.5
            b2x2 = tbx + tbw * 0.5
            b2y1 = tby - tbh * 0.5
            b2y2 = tby + tbh * 0.5
            inter = (jnp.maximum(jnp.minimum(b1x2, b2x2) - jnp.maximum(b1x1, b2x1), 0.0)
                     * jnp.maximum(jnp.minimum(b1y2, b2y2) - jnp.maximum(b1y1, b2y1), 0.0))
            w1 = pww
            h1 = phh + eps
            w2 = tbw
            h2 = tbh + eps
            union = w1 * h1 + w2 * h2 - inter + eps
            iou = inter / union
            cw = jnp.maximum(b1x2, b2x2) - jnp.minimum(b1x1, b2x1)
            ch = jnp.maximum(b1y2, b2y2) - jnp.minimum(b1y1, b2y1)
            c2 = cw * cw + ch * ch + eps
            dx = b2x1 + b2x2 - b1x1 - b1x2
            dy = b2y1 + b2y2 - b1y1 - b1y2
            rho2 = (dx * dx + dy * dy) * 0.25
            datan = _atan_pos(w2 / h2) - _atan_pos(w1 / h1)
            v = (4.0 / (math.pi ** 2)) * (datan * datan)
            alpha = v / (1.0 + eps - iou + v)
            iou_c = iou - (rho2 / c2 + v * alpha)
            l_acc = l_acc + validf * (1.0 - iou_c)
            c_acc = c_acc + validf
            o_acc = o_acc + validf * ps[4] * jnp.maximum(iou_c, 0.0)
            return (l_acc, c_acc, o_acc)

        zz = jnp.zeros((16,), jnp.float32)
        l_acc, c_acc, o_acc = lax.fori_loop(0, _KMAX, phase3, (zz, zz, zz))
        stage_v[lvl, 0] = l_acc
        stage_v[lvl, 1] = c_acc
        stage_v[lvl, 2] = o_acc

    pltpu.sync_copy(stage_v, out_hbm.at[wid])


def _sc_partials(targets_t, p0r, p1r, p2r):
    mesh = plsc.VectorSubcoreMesh(core_axis_name="c", subcore_axis_name="s")
    f = functools.partial(
        pl.kernel,
        mesh=mesh,
        out_type=jax.ShapeDtypeStruct((_NW, 3, 3, 16), jnp.float32),
        scratch_types=[
            pltpu.VMEM((6, _NT), jnp.float32),
            pltpu.VMEM((_KMAX, 16), jnp.int32),
            pltpu.VMEM((_KMAX, 16), jnp.float32),
            pltpu.VMEM((_KMAX, 16), jnp.float32),
            pltpu.VMEM((_KMAX, 16), jnp.float32),
            pltpu.VMEM((_KMAX, 16), jnp.float32),
            pltpu.VMEM((_KMAX, 16), jnp.float32),
            pltpu.VMEM((_KMAX, 16), jnp.float32),
            pltpu.VMEM((_KMAX, 16), jnp.float32),
            pltpu.VMEM((_KMAX, 16), jnp.float32),
            pltpu.VMEM((_KMAX, 16), jnp.float32),
            pltpu.VMEM((_KMAX, 16), jnp.float32),
            pltpu.VMEM((3, 3, 16), jnp.float32),
            pltpu.SemaphoreType.DMA,
        ],
    )(_sc_body)
    return f(targets_t, p0r, p1r, p2r)


@jax.jit
def kernel(p0, p1, p2, targets):
    sums = _dense_sums(p0, p1, p2)
    part = _sc_partials(
        targets.T,
        p0.reshape(-1),
        p1.reshape(-1),
        p2.reshape(-1),
    ).sum(axis=(0, 3))
    lbox = jnp.float32(0.0)
    lobj = jnp.float32(0.0)
    for i, (hh, ww) in enumerate(_DIMS):
        n = _B * _NA * hh * ww
        lbox = lbox + jnp.where(part[i, 1] > 0.0, part[i, 0] / part[i, 1], 0.0)
        lobj = lobj + (sums[i] - part[i, 2]) / n * _BAL[i]
    return (lbox.reshape(1), lobj.reshape(1), jnp.zeros(1, jnp.float32))


# speedup vs baseline: 1.6376x; 1.6376x over previous
import functools
import math

import jax
import jax.numpy as jnp
from jax import lax
from jax.experimental import pallas as pl
from jax.experimental.pallas import tpu as pltpu
from jax.experimental.pallas import tpu_sc as plsc

_B = 64
_NT = 800
_NA = 3
_THR = 4.0
_BAL = (4.0, 1.0, 0.4)
_DIMS = ((80, 80), (40, 40), (20, 20))
_STRIDE = (8.0, 16.0, 32.0)
_ANCHORS_RAW = ((10.0, 13.0, 16.0, 30.0, 33.0, 23.0),
                (30.0, 61.0, 62.0, 45.0, 59.0, 119.0),
                (116.0, 90.0, 156.0, 198.0, 373.0, 326.0))
_ANCH = tuple(
    tuple((_ANCHORS_RAW[i][2 * a] / _STRIDE[i], _ANCHORS_RAW[i][2 * a + 1] / _STRIDE[i])
          for a in range(_NA))
    for i in range(3))

_NW = 32
_LANES = 16
_TV = _NT // _LANES
_TASKS = 5 * _NA * _TV
_KMAX = (_TASKS + _NW - 1) // _NW

_G = 10


def _dense_body_one(ref, s_ref):
    @pl.when(pl.program_id(0) == 0)
    def _init():
        s_ref[0] = 0.0

    x4 = ref[:, :, :, :, 4]
    xd = x4.reshape(-1, 128)
    sp = jnp.maximum(xd, 0.0) + jnp.log(1.0 + jnp.exp(-jnp.abs(xd)))
    s_ref[0] += jnp.sum(sp)


def _dense_sums(p0, p1, p2):
    outs = []
    for p, nimg, (hh, ww) in ((p0, 1, (80, 80)), (p1, 4, (40, 40)),
                              (p2, 16, (20, 20))):
        out = pl.pallas_call(
            _dense_body_one,
            grid=(_B // nimg,),
            in_specs=[pl.BlockSpec((nimg, 3, hh, ww, 6),
                                   lambda b: (b, 0, 0, 0, 0))],
            out_specs=pl.BlockSpec(memory_space=pltpu.SMEM),
            out_shape=jax.ShapeDtypeStruct((1,), jnp.float32),
        )(p)
        outs.append(out)
    return jnp.concatenate(outs)


def _atan_pos(z):
    inv = z > 1.0
    zz = jnp.where(inv, 1.0 / z, z)
    u = zz * zz
    p = 0.99997726 + u * (-0.33262347 + u * (0.19354346 + u * (
        -0.11643287 + u * (0.05265332 + u * (-0.01172120)))))
    a = zz * p
    return jnp.where(inv, (math.pi / 2) - a, a)


def _sigmoid(z):
    return 1.0 / (1.0 + jnp.exp(-z))


def _sc_body(tt_hbm, p0_hbm, p1_hbm, p2_hbm, out_hbm,
             tt_v, row_v, val_v, tbx_v, tby_v, tbw_v, tbh_v,
             psa_v, psb_v, psc_v, psd_v, pse_v, stage_v, sem):
    ps_v = (psa_v, psb_v, psc_v, psd_v, pse_v)
    wid = lax.axis_index("s") * 2 + lax.axis_index("c")
    pltpu.sync_copy(tt_hbm, tt_v)
    lanes = lax.iota(jnp.int32, 16)
    flanes = lanes.astype(jnp.float32)

    def task_decode(k):
        task = wid + _NW * k
        offi = jnp.minimum(lax.div(task, _NA * _TV), 4)
        rem = lax.rem(task, _NA * _TV)
        ai = lax.div(rem, _TV)
        tv = lax.rem(rem, _TV)
        padf = jnp.where(task >= _TASKS, 1.0, 0.0)
        return task, offi, ai, tv, padf

    def sel_anchor(lvl, ai):
        aw = jnp.where(ai == 0, _ANCH[lvl][0][0],
                       jnp.where(ai == 1, _ANCH[lvl][1][0], _ANCH[lvl][2][0]))
        ah = jnp.where(ai == 0, _ANCH[lvl][0][1],
                       jnp.where(ai == 1, _ANCH[lvl][1][1], _ANCH[lvl][2][1]))
        return aw, ah

    for lvl, (p_hbm, (hh, ww)) in enumerate(
            zip((p0_hbm, p1_hbm, p2_hbm), _DIMS)):
        fw = float(ww)
        fh = float(hh)

        def phase1(k, carry, _fw=fw, _fh=fh, _lvl=lvl):
            task, offi, ai, tv, padf = task_decode(k)
            base = tv * _LANES
            bi = tt_v[0, pl.ds(base, 16)]
            x = tt_v[2, pl.ds(base, 16)]
            y = tt_v[3, pl.ds(base, 16)]
            w = tt_v[4, pl.ds(base, 16)]
            h = tt_v[5, pl.ds(base, 16)]
            tx = x * _fw
            ty = y * _fh
            tw = w * _fw
            th = h * _fh
            aw, ah = sel_anchor(_lvl, ai)
            rw = tw / aw
            rh = th / ah
            rj = jnp.maximum(jnp.maximum(rw, 1.0 / rw),
                             jnp.maximum(rh, 1.0 / rh)) < _THR
            jjf = jnp.where((lax.rem(tx, 1.0) < 0.5) & (tx > 1.0), 1.0, 0.0)
            kkf = jnp.where((lax.rem(ty, 1.0) < 0.5) & (ty > 1.0), 1.0, 0.0)
            gxi = _fw - tx
            gyi = _fh - ty
            llf = jnp.where((lax.rem(gxi, 1.0) < 0.5) & (gxi > 1.0), 1.0, 0.0)
            mmf = jnp.where((lax.rem(gyi, 1.0) < 0.5) & (gyi > 1.0), 1.0, 0.0)
            e0 = jnp.where(offi == 0, 1.0, 0.0)
            e1 = jnp.where(offi == 1, 1.0, 0.0)
            e2 = jnp.where(offi == 2, 1.0, 0.0)
            e3 = jnp.where(offi == 3, 1.0, 0.0)
            e4 = jnp.where(offi == 4, 1.0, 0.0)
            jmf = e0 + e1 * jjf + e2 * kkf + e3 * llf + e4 * mmf
            rjf = jnp.where(rj, 1.0, 0.0)
            validf = jmf * rjf * (1.0 - padf)
            offx = jnp.where(offi == 1, 0.5, 0.0) + jnp.where(offi == 3, -0.5, 0.0)
            offy = jnp.where(offi == 2, 0.5, 0.0) + jnp.where(offi == 4, -0.5, 0.0)
            gxf = tx - offx
            gyf = ty - offy
            gijx = gxf.astype(jnp.int32)
            gijy = gyf.astype(jnp.int32)
            gi = jnp.clip(gijx, 0, ww - 1)
            gj = jnp.clip(gijy, 0, hh - 1)
            b = bi.astype(jnp.int32)
            row = ((b * _NA + ai) * hh + gj) * ww + gi
            row_v[k] = row
            val_v[k] = validf
            tbx_v[k] = tx - gijx.astype(jnp.float32)
            tby_v[k] = ty - gijy.astype(jnp.float32)
            tbw_v[k] = tw
            tbh_v[k] = th
            return carry

        lax.fori_loop(0, _KMAX, phase1, 0)

        copies = []
        for k in range(_KMAX):
            idx6 = row_v[k] * 6
            for c in range(5):
                copies.append(pltpu.make_async_copy(
                    p_hbm.at[idx6 + c], ps_v[c].at[k], sem))
        for cp in copies:
            cp.start()
        for cp in copies:
            cp.wait()

        def phase3(k, carry, _lvl=lvl):
            l_acc, c_acc, o_acc = carry
            task, offi, ai, tv, padf = task_decode(k)
            ps = [ps_v[c][k] for c in range(5)]
            validf = val_v[k]
            tbx = tbx_v[k]
            tby = tby_v[k]
            tbw = tbw_v[k]
            tbh = tbh_v[k]
            aw, ah = sel_anchor(_lvl, ai)
            pxx = _sigmoid(ps[0]) * 2.0 - 0.5
            pxy = _sigmoid(ps[1]) * 2.0 - 0.5
            sw = _sigmoid(ps[2]) * 2.0
            sh = _sigmoid(ps[3]) * 2.0
            pww = sw * sw * aw
            phh = sh * sh * ah
            eps = 1e-9
            b1x1 = pxx - pww * 0.5
            b1x2 = pxx + pww * 0.5
            b1y1 = pxy - phh * 0.5
            b1y2 = pxy + phh * 0.5
            b2x1 = tbx - tbw * 0.5
            b2x2 = tbx + tbw * 0.5
            b2y1 = tby - tbh * 0.5
            b2y2 = tby + tbh * 0.5
            inter = (jnp.maximum(jnp.minimum(b1x2, b2x2) - jnp.maximum(b1x1, b2x1), 0.0)
                     * jnp.maximum(jnp.minimum(b1y2, b2y2) - jnp.maximum(b1y1, b2y1), 0.0))
            w1 = pww
            h1 = phh + eps
            w2 = tbw
            h2 = tbh + eps
            union = w1 * h1 + w2 * h2 - inter + eps
            iou = inter / union
            cw = jnp.maximum(b1x2, b2x2) - jnp.minimum(b1x1, b2x1)
            ch = jnp.maximum(b1y2, b2y2) - jnp.minimum(b1y1, b2y1)
            c2 = cw * cw + ch * ch + eps
            dx = b2x1 + b2x2 - b1x1 - b1x2
            dy = b2y1 + b2y2 - b1y1 - b1y2
            rho2 = (dx * dx + dy * dy) * 0.25
            datan = _atan_pos(w2 / h2) - _atan_pos(w1 / h1)
            v = (4.0 / (math.pi ** 2)) * (datan * datan)
            alpha = v / (1.0 + eps - iou + v)
            iou_c = iou - (rho2 / c2 + v * alpha)
            l_acc = l_acc + validf * (1.0 - iou_c)
            c_acc = c_acc + validf
            o_acc = o_acc + validf * ps[4] * jnp.maximum(iou_c, 0.0)
            return (l_acc, c_acc, o_acc)

        zz = jnp.zeros((16,), jnp.float32)
        l_acc, c_acc, o_acc = lax.fori_loop(0, _KMAX, phase3, (zz, zz, zz))
        stage_v[lvl, 0] = l_acc
        stage_v[lvl, 1] = c_acc
        stage_v[lvl, 2] = o_acc

    pltpu.sync_copy(stage_v, out_hbm.at[wid])


def _sc_partials(targets_t, p0r, p1r, p2r):
    mesh = plsc.VectorSubcoreMesh(core_axis_name="c", subcore_axis_name="s")
    f = functools.partial(
        pl.kernel,
        mesh=mesh,
        out_type=jax.ShapeDtypeStruct((_NW, 3, 3, 16), jnp.float32),
        scratch_types=[
            pltpu.VMEM((6, _NT), jnp.float32),
            pltpu.VMEM((_KMAX, 16), jnp.int32),
            pltpu.VMEM((_KMAX, 16), jnp.float32),
            pltpu.VMEM((_KMAX, 16), jnp.float32),
            pltpu.VMEM((_KMAX, 16), jnp.float32),
            pltpu.VMEM((_KMAX, 16), jnp.float32),
            pltpu.VMEM((_KMAX, 16), jnp.float32),
            pltpu.VMEM((_KMAX, 16), jnp.float32),
            pltpu.VMEM((_KMAX, 16), jnp.float32),
            pltpu.VMEM((_KMAX, 16), jnp.float32),
            pltpu.VMEM((_KMAX, 16), jnp.float32),
            pltpu.VMEM((_KMAX, 16), jnp.float32),
            pltpu.VMEM((3, 3, 16), jnp.float32),
            pltpu.SemaphoreType.DMA,
        ],
    )(_sc_body)
    return f(targets_t, p0r, p1r, p2r)


@jax.jit
def kernel(p0, p1, p2, targets):
    sums = _dense_sums(p0, p1, p2)
    part = _sc_partials(
        targets.T,
        p0.reshape(-1),
        p1.reshape(-1),
        p2.reshape(-1),
    ).sum(axis=(0, 3))
    lbox = jnp.float32(0.0)
    lobj = jnp.float32(0.0)
    for i, (hh, ww) in enumerate(_DIMS):
        n = _B * _NA * hh * ww
        lbox = lbox + jnp.where(part[i, 1] > 0.0, part[i, 0] / part[i, 1], 0.0)
        lobj = lobj + (sums[i] - part[i, 2]) / n * _BAL[i]
    return (lbox.reshape(1), lobj.reshape(1), jnp.zeros(1, jnp.float32))


# speedup vs baseline: 2.9758x; 1.8172x over previous
import functools
import math

import jax
import jax.numpy as jnp
from jax import lax
from jax.experimental import pallas as pl
from jax.experimental.pallas import tpu as pltpu
from jax.experimental.pallas import tpu_sc as plsc

_B = 64
_NT = 800
_NA = 3
_THR = 4.0
_BAL = (4.0, 1.0, 0.4)
_DIMS = ((80, 80), (40, 40), (20, 20))
_STRIDE = (8.0, 16.0, 32.0)
_ANCHORS_RAW = ((10.0, 13.0, 16.0, 30.0, 33.0, 23.0),
                (30.0, 61.0, 62.0, 45.0, 59.0, 119.0),
                (116.0, 90.0, 156.0, 198.0, 373.0, 326.0))
_ANCH = tuple(
    tuple((_ANCHORS_RAW[i][2 * a] / _STRIDE[i], _ANCHORS_RAW[i][2 * a + 1] / _STRIDE[i])
          for a in range(_NA))
    for i in range(3))

_NW = 32
_LANES = 16
_TV = _NT // _LANES
_TASKS = 5 * _NA * _TV
_KMAX = (_TASKS + _NW - 1) // _NW

_G = 10


def _dense_body_one(ref, s_ref):
    @pl.when(pl.program_id(0) == 0)
    def _init():
        s_ref[0] = 0.0

    x4 = ref[:, :, :, :, 4]
    xd = x4.reshape(-1, 128)
    sp = jnp.maximum(xd, 0.0) + jnp.log(1.0 + jnp.exp(-jnp.abs(xd)))
    s_ref[0] += jnp.sum(sp)


def _dense_sums(p0, p1, p2):
    outs = []
    for p, nimg, (hh, ww) in ((p0, 1, (80, 80)), (p1, 4, (40, 40)),
                              (p2, 16, (20, 20))):
        out = pl.pallas_call(
            _dense_body_one,
            grid=(_B // nimg,),
            in_specs=[pl.BlockSpec((nimg, 3, hh, ww, 6),
                                   lambda b: (b, 0, 0, 0, 0))],
            out_specs=pl.BlockSpec(memory_space=pltpu.SMEM),
            out_shape=jax.ShapeDtypeStruct((1,), jnp.float32),
        )(p)
        outs.append(out)
    return jnp.concatenate(outs)


def _atan_pos(z):
    inv = z > 1.0
    zz = jnp.where(inv, 1.0 / z, z)
    u = zz * zz
    p = 0.99997726 + u * (-0.33262347 + u * (0.19354346 + u * (
        -0.11643287 + u * (0.05265332 + u * (-0.01172120)))))
    a = zz * p
    return jnp.where(inv, (math.pi / 2) - a, a)


def _sigmoid(z):
    return 1.0 / (1.0 + jnp.exp(-z))


def _sc_body(tt_hbm, p0_hbm, p1_hbm, p2_hbm, out_hbm,
             tt_v, key_v, val_v, tbx_v, tby_v, tbw_v, tbh_v,
             ps_v, stage_v, sem):
    wid = lax.axis_index("s") * 2 + lax.axis_index("c")
    pltpu.sync_copy(tt_hbm, tt_v)
    lanes = lax.iota(jnp.int32, 16)
    flanes = lanes.astype(jnp.float32)

    def task_decode(k):
        task = wid + _NW * k
        offi = jnp.minimum(lax.div(task, _NA * _TV), 4)
        rem = lax.rem(task, _NA * _TV)
        ai = lax.div(rem, _TV)
        tv = lax.rem(rem, _TV)
        padf = jnp.where(task >= _TASKS, 1.0, 0.0)
        return task, offi, ai, tv, padf

    def sel_anchor(lvl, ai):
        aw = jnp.where(ai == 0, _ANCH[lvl][0][0],
                       jnp.where(ai == 1, _ANCH[lvl][1][0], _ANCH[lvl][2][0]))
        ah = jnp.where(ai == 0, _ANCH[lvl][0][1],
                       jnp.where(ai == 1, _ANCH[lvl][1][1], _ANCH[lvl][2][1]))
        return aw, ah

    for lvl, (p_hbm, (hh, ww)) in enumerate(
            zip((p0_hbm, p1_hbm, p2_hbm), _DIMS)):
        fw = float(ww)
        fh = float(hh)

        def phase1(k, carry, _fw=fw, _fh=fh, _lvl=lvl):
            task, offi, ai, tv, padf = task_decode(k)
            base = tv * _LANES
            bi = tt_v[0, pl.ds(base, 16)]
            x = tt_v[2, pl.ds(base, 16)]
            y = tt_v[3, pl.ds(base, 16)]
            w = tt_v[4, pl.ds(base, 16)]
            h = tt_v[5, pl.ds(base, 16)]
            tx = x * _fw
            ty = y * _fh
            tw = w * _fw
            th = h * _fh
            aw, ah = sel_anchor(_lvl, ai)
            rw = tw / aw
            rh = th / ah
            rj = jnp.maximum(jnp.maximum(rw, 1.0 / rw),
                             jnp.maximum(rh, 1.0 / rh)) < _THR
            jjf = jnp.where((lax.rem(tx, 1.0) < 0.5) & (tx > 1.0), 1.0, 0.0)
            kkf = jnp.where((lax.rem(ty, 1.0) < 0.5) & (ty > 1.0), 1.0, 0.0)
            gxi = _fw - tx
            gyi = _fh - ty
            llf = jnp.where((lax.rem(gxi, 1.0) < 0.5) & (gxi > 1.0), 1.0, 0.0)
            mmf = jnp.where((lax.rem(gyi, 1.0) < 0.5) & (gyi > 1.0), 1.0, 0.0)
            e0 = jnp.where(offi == 0, 1.0, 0.0)
            e1 = jnp.where(offi == 1, 1.0, 0.0)
            e2 = jnp.where(offi == 2, 1.0, 0.0)
            e3 = jnp.where(offi == 3, 1.0, 0.0)
            e4 = jnp.where(offi == 4, 1.0, 0.0)
            jmf = e0 + e1 * jjf + e2 * kkf + e3 * llf + e4 * mmf
            rjf = jnp.where(rj, 1.0, 0.0)
            validf = jmf * rjf * (1.0 - padf)
            offx = jnp.where(offi == 1, 0.5, 0.0) + jnp.where(offi == 3, -0.5, 0.0)
            offy = jnp.where(offi == 2, 0.5, 0.0) + jnp.where(offi == 4, -0.5, 0.0)
            gxf = tx - offx
            gyf = ty - offy
            gijx = gxf.astype(jnp.int32)
            gijy = gyf.astype(jnp.int32)
            gi = jnp.clip(gijx, 0, ww - 1)
            gj = jnp.clip(gijy, 0, hh - 1)
            b = bi.astype(jnp.int32)
            sl = pl.ds(k * 16, 16)
            key_v[sl] = (b * hh + gj) * ww + gi
            val_v[sl] = validf
            tbx_v[sl] = tx - gijx.astype(jnp.float32)
            tby_v[sl] = ty - gijy.astype(jnp.float32)
            tbw_v[sl] = tw
            tbh_v[sl] = th
            return carry

        lax.fori_loop(0, _KMAX, phase1, 0)

        def phase2(k, carry, _p_hbm=p_hbm, _hh=hh, _ww=ww):
            task, offi, ai, tv, padf = task_decode(k)
            kv = key_v[pl.ds(k * 16, 16)]
            cps = []
            for j in range(16):
                key = kv[j]
                gii = lax.rem(key, _ww)
                t = lax.div(key, _ww)
                gjj = lax.rem(t, _hh)
                bj = lax.div(t, _hh)
                cp = pltpu.make_async_copy(
                    _p_hbm.at[bj, ai, gjj, gii],
                    ps_v.at[k * 16 + j, pl.ds(0, 6)], sem)
                cp.start()
                cps.append(cp)
            for cp in cps:
                cp.wait()
            return carry

        lax.fori_loop(0, _KMAX, phase2, 0)

        def phase3(k, carry, _lvl=lvl):
            l_acc, c_acc, o_acc = carry
            task, offi, ai, tv, padf = task_decode(k)
            zf = jnp.zeros((16,), jnp.float32)
            ps = [zf] * 5
            for j in range(16):
                rv = ps_v[k * 16 + j, pl.ds(0, 16)]
                lm = lanes == j
                for c in range(5):
                    ps[c] = jnp.where(lm, rv[c], ps[c])
            sl = pl.ds(k * 16, 16)
            validf = val_v[sl]
            tbx = tbx_v[sl]
            tby = tby_v[sl]
            tbw = tbw_v[sl]
            tbh = tbh_v[sl]
            aw, ah = sel_anchor(_lvl, ai)
            pxx = _sigmoid(ps[0]) * 2.0 - 0.5
            pxy = _sigmoid(ps[1]) * 2.0 - 0.5
            sw = _sigmoid(ps[2]) * 2.0
            sh = _sigmoid(ps[3]) * 2.0
            pww = sw * sw * aw
            phh = sh * sh * ah
            eps = 1e-9
            b1x1 = pxx - pww * 0.5
            b1x2 = pxx + pww * 0.5
            b1y1 = pxy - phh * 0.5
            b1y2 = pxy + phh * 0.5
            b2x1 = tbx - tbw * 0.5
            b2x2 = tbx + tbw * 0.5
            b2y1 = tby - tbh * 0.5
            b2y2 = tby + tbh * 0.5
            inter = (jnp.maximum(jnp.minimum(b1x2, b2x2) - jnp.maximum(b1x1, b2x1), 0.0)
                     * jnp.maximum(jnp.minimum(b1y2, b2y2) - jnp.maximum(b1y1, b2y1), 0.0))
            w1 = pww
            h1 = phh + eps
            w2 = tbw
            h2 = tbh + eps
            union = w1 * h1 + w2 * h2 - inter + eps
            iou = inter / union
            cw = jnp.maximum(b1x2, b2x2) - jnp.minimum(b1x1, b2x1)
            ch = jnp.maximum(b1y2, b2y2) - jnp.minimum(b1y1, b2y1)
            c2 = cw * cw + ch * ch + eps
            dx = b2x1 + b2x2 - b1x1 - b1x2
            dy = b2y1 + b2y2 - b1y1 - b1y2
            rho2 = (dx * dx + dy * dy) * 0.25
            datan = _atan_pos(w2 / h2) - _atan_pos(w1 / h1)
            v = (4.0 / (math.pi ** 2)) * (datan * datan)
            alpha = v / (1.0 + eps - iou + v)
            iou_c = iou - (rho2 / c2 + v * alpha)
            l_acc = l_acc + validf * (1.0 - iou_c)
            c_acc = c_acc + validf
            o_acc = o_acc + validf * ps[4] * jnp.maximum(iou_c, 0.0)
            return (l_acc, c_acc, o_acc)

        zz = jnp.zeros((16,), jnp.float32)
        l_acc, c_acc, o_acc = lax.fori_loop(0, _KMAX, phase3, (zz, zz, zz))
        stage_v[pl.ds((lvl * 3 + 0) * 16, 16)] = l_acc
        stage_v[pl.ds((lvl * 3 + 1) * 16, 16)] = c_acc
        stage_v[pl.ds((lvl * 3 + 2) * 16, 16)] = o_acc

    pltpu.sync_copy(stage_v, out_hbm.at[wid])


_NCAND = _KMAX * 16


def _sc_partials(targets_t, p0, p1, p2):
    mesh = plsc.VectorSubcoreMesh(core_axis_name="c", subcore_axis_name="s")
    f = functools.partial(
        pl.kernel,
        mesh=mesh,
        out_type=jax.ShapeDtypeStruct((_NW, 144), jnp.float32),
        compiler_params=pltpu.CompilerParams(use_tc_tiling_on_sc=True),
        scratch_types=[
            pltpu.VMEM((6, _NT), jnp.float32),
            pltpu.VMEM((_NCAND,), jnp.int32),
            pltpu.VMEM((_NCAND,), jnp.float32),
            pltpu.VMEM((_NCAND,), jnp.float32),
            pltpu.VMEM((_NCAND,), jnp.float32),
            pltpu.VMEM((_NCAND,), jnp.float32),
            pltpu.VMEM((_NCAND,), jnp.float32),
            pltpu.VMEM((_NCAND, 128), jnp.float32),
            pltpu.VMEM((144,), jnp.float32),
            pltpu.SemaphoreType.DMA,
        ],
    )(_sc_body)
    return f(targets_t, p0, p1, p2)


@jax.jit
def kernel(p0, p1, p2, targets):
    sums = _dense_sums(p0, p1, p2)
    part = _sc_partials(targets.T, p0, p1, p2)
    part = part.reshape(_NW, 3, 3, 16).sum(axis=(0, 3))
    lbox = jnp.float32(0.0)
    lobj = jnp.float32(0.0)
    for i, (hh, ww) in enumerate(_DIMS):
        n = _B * _NA * hh * ww
        lbox = lbox + jnp.where(part[i, 1] > 0.0, part[i, 0] / part[i, 1], 0.0)
        lobj = lobj + (sums[i] - part[i, 2]) / n * _BAL[i]
    return (lbox.reshape(1), lobj.reshape(1), jnp.zeros(1, jnp.float32))


# speedup vs baseline: 2.9851x; 1.0031x over previous
import functools
import math

import jax
import jax.numpy as jnp
from jax import lax
from jax.experimental import pallas as pl
from jax.experimental.pallas import tpu as pltpu
from jax.experimental.pallas import tpu_sc as plsc

_B = 64
_NT = 800
_NA = 3
_THR = 4.0
_BAL = (4.0, 1.0, 0.4)
_DIMS = ((80, 80), (40, 40), (20, 20))
_STRIDE = (8.0, 16.0, 32.0)
_ANCHORS_RAW = ((10.0, 13.0, 16.0, 30.0, 33.0, 23.0),
                (30.0, 61.0, 62.0, 45.0, 59.0, 119.0),
                (116.0, 90.0, 156.0, 198.0, 373.0, 326.0))
_ANCH = tuple(
    tuple((_ANCHORS_RAW[i][2 * a] / _STRIDE[i], _ANCHORS_RAW[i][2 * a + 1] / _STRIDE[i])
          for a in range(_NA))
    for i in range(3))

_NW = 32
_LANES = 16
_TV = _NT // _LANES
_TASKS = 5 * _NA * _TV
_KMAX = (_TASKS + _NW - 1) // _NW

_G = 10


def _dense_body_one(ref, s_ref):
    @pl.when(pl.program_id(0) == 0)
    def _init():
        s_ref[0] = 0.0

    x4 = ref[:, :, :, :, 4]
    xd = x4.reshape(-1, 128)
    sp = jnp.maximum(xd, 0.0) + jnp.log(1.0 + jnp.exp(-jnp.abs(xd)))
    s_ref[0] += jnp.sum(sp)


def _dense_sums(p0, p1, p2):
    outs = []
    for p, nimg, (hh, ww) in ((p0, 2, (80, 80)), (p1, 8, (40, 40)),
                              (p2, 32, (20, 20))):
        out = pl.pallas_call(
            _dense_body_one,
            grid=(_B // nimg,),
            in_specs=[pl.BlockSpec((nimg, 3, hh, ww, 6),
                                   lambda b: (b, 0, 0, 0, 0))],
            out_specs=pl.BlockSpec(memory_space=pltpu.SMEM),
            out_shape=jax.ShapeDtypeStruct((1,), jnp.float32),
        )(p)
        outs.append(out)
    return jnp.concatenate(outs)


def _atan_pos(z):
    inv = z > 1.0
    zz = jnp.where(inv, 1.0 / z, z)
    u = zz * zz
    p = 0.99997726 + u * (-0.33262347 + u * (0.19354346 + u * (
        -0.11643287 + u * (0.05265332 + u * (-0.01172120)))))
    a = zz * p
    return jnp.where(inv, (math.pi / 2) - a, a)


def _sigmoid(z):
    return 1.0 / (1.0 + jnp.exp(-z))


def _sc_body(tt_hbm, p0_hbm, p1_hbm, p2_hbm, out_hbm,
             tt_v, key_v, val_v, tbx_v, tby_v, tbw_v, tbh_v,
             ps_v, stage_v, sem):
    wid = lax.axis_index("s") * 2 + lax.axis_index("c")
    pltpu.sync_copy(tt_hbm, tt_v)
    lanes = lax.iota(jnp.int32, 16)
    flanes = lanes.astype(jnp.float32)

    def task_decode(k):
        task = wid + _NW * k
        offi = jnp.minimum(lax.div(task, _NA * _TV), 4)
        rem = lax.rem(task, _NA * _TV)
        ai = lax.div(rem, _TV)
        tv = lax.rem(rem, _TV)
        padf = jnp.where(task >= _TASKS, 1.0, 0.0)
        return task, offi, ai, tv, padf

    def sel_anchor(lvl, ai):
        aw = jnp.where(ai == 0, _ANCH[lvl][0][0],
                       jnp.where(ai == 1, _ANCH[lvl][1][0], _ANCH[lvl][2][0]))
        ah = jnp.where(ai == 0, _ANCH[lvl][0][1],
                       jnp.where(ai == 1, _ANCH[lvl][1][1], _ANCH[lvl][2][1]))
        return aw, ah

    for lvl, (p_hbm, (hh, ww)) in enumerate(
            zip((p0_hbm, p1_hbm, p2_hbm), _DIMS)):
        fw = float(ww)
        fh = float(hh)

        def phase1(k, carry, _fw=fw, _fh=fh, _lvl=lvl):
            task, offi, ai, tv, padf = task_decode(k)
            base = tv * _LANES
            bi = tt_v[0, pl.ds(base, 16)]
            x = tt_v[2, pl.ds(base, 16)]
            y = tt_v[3, pl.ds(base, 16)]
            w = tt_v[4, pl.ds(base, 16)]
            h = tt_v[5, pl.ds(base, 16)]
            tx = x * _fw
            ty = y * _fh
            tw = w * _fw
            th = h * _fh
            aw, ah = sel_anchor(_lvl, ai)
            rw = tw / aw
            rh = th / ah
            rj = jnp.maximum(jnp.maximum(rw, 1.0 / rw),
                             jnp.maximum(rh, 1.0 / rh)) < _THR
            jjf = jnp.where((lax.rem(tx, 1.0) < 0.5) & (tx > 1.0), 1.0, 0.0)
            kkf = jnp.where((lax.rem(ty, 1.0) < 0.5) & (ty > 1.0), 1.0, 0.0)
            gxi = _fw - tx
            gyi = _fh - ty
            llf = jnp.where((lax.rem(gxi, 1.0) < 0.5) & (gxi > 1.0), 1.0, 0.0)
            mmf = jnp.where((lax.rem(gyi, 1.0) < 0.5) & (gyi > 1.0), 1.0, 0.0)
            e0 = jnp.where(offi == 0, 1.0, 0.0)
            e1 = jnp.where(offi == 1, 1.0, 0.0)
            e2 = jnp.where(offi == 2, 1.0, 0.0)
            e3 = jnp.where(offi == 3, 1.0, 0.0)
            e4 = jnp.where(offi == 4, 1.0, 0.0)
            jmf = e0 + e1 * jjf + e2 * kkf + e3 * llf + e4 * mmf
            rjf = jnp.where(rj, 1.0, 0.0)
            validf = jmf * rjf * (1.0 - padf)
            offx = jnp.where(offi == 1, 0.5, 0.0) + jnp.where(offi == 3, -0.5, 0.0)
            offy = jnp.where(offi == 2, 0.5, 0.0) + jnp.where(offi == 4, -0.5, 0.0)
            gxf = tx - offx
            gyf = ty - offy
            gijx = gxf.astype(jnp.int32)
            gijy = gyf.astype(jnp.int32)
            gi = jnp.clip(gijx, 0, ww - 1)
            gj = jnp.clip(gijy, 0, hh - 1)
            b = bi.astype(jnp.int32)
            sl = pl.ds(k * 16, 16)
            key_v[sl] = (b * hh + gj) * ww + gi
            val_v[sl] = validf
            tbx_v[sl] = tx - gijx.astype(jnp.float32)
            tby_v[sl] = ty - gijy.astype(jnp.float32)
            tbw_v[sl] = tw
            tbh_v[sl] = th
            return carry

        lax.fori_loop(0, _KMAX, phase1, 0)

        def phase2(k, carry, _p_hbm=p_hbm, _hh=hh, _ww=ww):
            task, offi, ai, tv, padf = task_decode(k)
            kv = key_v[pl.ds(k * 16, 16)]
            cps = []
            for j in range(16):
                key = kv[j]
                gii = lax.rem(key, _ww)
                t = lax.div(key, _ww)
                gjj = lax.rem(t, _hh)
                bj = lax.div(t, _hh)
                cp = pltpu.make_async_copy(
                    _p_hbm.at[bj, ai, gjj, gii],
                    ps_v.at[k * 16 + j, pl.ds(0, 6)], sem)
                cp.start()
                cps.append(cp)
            for cp in cps:
                cp.wait()
            return carry

        lax.fori_loop(0, _KMAX, phase2, 0)

        def phase3(k, carry, _lvl=lvl):
            l_acc, c_acc, o_acc = carry
            task, offi, ai, tv, padf = task_decode(k)
            zf = jnp.zeros((16,), jnp.float32)
            ps = [zf] * 5
            for j in range(16):
                rv = ps_v[k * 16 + j, pl.ds(0, 16)]
                lm = lanes == j
                for c in range(5):
                    ps[c] = jnp.where(lm, rv[c], ps[c])
            sl = pl.ds(k * 16, 16)
            validf = val_v[sl]
            tbx = tbx_v[sl]
            tby = tby_v[sl]
            tbw = tbw_v[sl]
            tbh = tbh_v[sl]
            aw, ah = sel_anchor(_lvl, ai)
            pxx = _sigmoid(ps[0]) * 2.0 - 0.5
            pxy = _sigmoid(ps[1]) * 2.0 - 0.5
            sw = _sigmoid(ps[2]) * 2.0
            sh = _sigmoid(ps[3]) * 2.0
            pww = sw * sw * aw
            phh = sh * sh * ah
            eps = 1e-9
            b1x1 = pxx - pww * 0.5
            b1x2 = pxx + pww * 0.5
            b1y1 = pxy - phh * 0.5
            b1y2 = pxy + phh * 0.5
            b2x1 = tbx - tbw * 0.5
            b2x2 = tbx + tbw * 0.5
            b2y1 = tby - tbh * 0.5
            b2y2 = tby + tbh * 0.5
            inter = (jnp.maximum(jnp.minimum(b1x2, b2x2) - jnp.maximum(b1x1, b2x1), 0.0)
                     * jnp.maximum(jnp.minimum(b1y2, b2y2) - jnp.maximum(b1y1, b2y1), 0.0))
            w1 = pww
            h1 = phh + eps
            w2 = tbw
            h2 = tbh + eps
            union = w1 * h1 + w2 * h2 - inter + eps
            iou = inter / union
            cw = jnp.maximum(b1x2, b2x2) - jnp.minimum(b1x1, b2x1)
            ch = jnp.maximum(b1y2, b2y2) - jnp.minimum(b1y1, b2y1)
            c2 = cw * cw + ch * ch + eps
            dx = b2x1 + b2x2 - b1x1 - b1x2
            dy = b2y1 + b2y2 - b1y1 - b1y2
            rho2 = (dx * dx + dy * dy) * 0.25
            datan = _atan_pos(w2 / h2) - _atan_pos(w1 / h1)
            v = (4.0 / (math.pi ** 2)) * (datan * datan)
            alpha = v / (1.0 + eps - iou + v)
            iou_c = iou - (rho2 / c2 + v * alpha)
            l_acc = l_acc + validf * (1.0 - iou_c)
            c_acc = c_acc + validf
            o_acc = o_acc + validf * ps[4] * jnp.maximum(iou_c, 0.0)
            return (l_acc, c_acc, o_acc)

        zz = jnp.zeros((16,), jnp.float32)
        l_acc, c_acc, o_acc = lax.fori_loop(0, _KMAX, phase3, (zz, zz, zz))
        stage_v[pl.ds((lvl * 3 + 0) * 16, 16)] = l_acc
        stage_v[pl.ds((lvl * 3 + 1) * 16, 16)] = c_acc
        stage_v[pl.ds((lvl * 3 + 2) * 16, 16)] = o_acc

    pltpu.sync_copy(stage_v, out_hbm.at[wid])


_NCAND = _KMAX * 16


def _sc_partials(targets_t, p0, p1, p2):
    mesh = plsc.VectorSubcoreMesh(core_axis_name="c", subcore_axis_name="s")
    f = functools.partial(
        pl.kernel,
        mesh=mesh,
        out_type=jax.ShapeDtypeStruct((_NW, 144), jnp.float32),
        compiler_params=pltpu.CompilerParams(use_tc_tiling_on_sc=True),
        scratch_types=[
            pltpu.VMEM((6, _NT), jnp.float32),
            pltpu.VMEM((_NCAND,), jnp.int32),
            pltpu.VMEM((_NCAND,), jnp.float32),
            pltpu.VMEM((_NCAND,), jnp.float32),
            pltpu.VMEM((_NCAND,), jnp.float32),
            pltpu.VMEM((_NCAND,), jnp.float32),
            pltpu.VMEM((_NCAND,), jnp.float32),
            pltpu.VMEM((_NCAND, 128), jnp.float32),
            pltpu.VMEM((144,), jnp.float32),
            pltpu.SemaphoreType.DMA,
        ],
    )(_sc_body)
    return f(targets_t, p0, p1, p2)


@jax.jit
def kernel(p0, p1, p2, targets):
    sums = _dense_sums(p0, p1, p2)
    part = _sc_partials(targets.T, p0, p1, p2)
    part = part.reshape(_NW, 3, 3, 16).sum(axis=(0, 3))
    lbox = jnp.float32(0.0)
    lobj = jnp.float32(0.0)
    for i, (hh, ww) in enumerate(_DIMS):
        n = _B * _NA * hh * ww
        lbox = lbox + jnp.where(part[i, 1] > 0.0, part[i, 0] / part[i, 1], 0.0)
        lobj = lobj + (sums[i] - part[i, 2]) / n * _BAL[i]
    return (lbox.reshape(1), lobj.reshape(1), jnp.zeros(1, jnp.float32))
